# Initial kernel scaffold; baseline (speedup 1.0000x reference)
import jax
import jax.numpy as jnp
from jax.experimental import pallas as pl


def kernel(val):
    raise NotImplementedError("write your pallas kernel here")


# speedup vs baseline: 1.8727x; 1.8727x over previous
import functools

import jax
import jax.numpy as jnp
from jax import lax
from jax.experimental import pallas as pl
from jax.experimental.pallas import tpu as pltpu
from jax.experimental.pallas import tpu_sc as plsc

B, S, D = 4, 4096, 2048
NC, NS = 2, 16
NW = NC * NS
DW = 128
ND = D // DW
NT = B * ND
TPW = NT // NW
NV = DW // 16
S_CHUNK = 256
N_CHUNK = S // S_CHUNK


def _cumsum_body(val_hbm, out_hbm, in_v, out_v):
    wid = lax.axis_index("s") * NC + lax.axis_index("c")
    for t in range(TPW):
        task = wid * TPW + t
        b = task // ND
        d0 = (task % ND) * DW
        carry = tuple(jnp.zeros((16,), jnp.float32) for _ in range(NV))
        for c in range(N_CHUNK):
            s0 = c * S_CHUNK
            pltpu.sync_copy(
                val_hbm.at[b, pl.ds(s0, S_CHUNK), pl.ds(d0, DW)], in_v)

            def row(s, carry):
                new = []
                for j in range(NV):
                    x = in_v[s, pl.ds(j * 16, 16)]
                    acc = carry[j] + x
                    out_v[s, pl.ds(j * 16, 16)] = acc
                    new.append(acc)
                return tuple(new)

            carry = lax.fori_loop(0, S_CHUNK, row, carry)
            pltpu.sync_copy(
                out_v, out_hbm.at[b, pl.ds(s0, S_CHUNK), pl.ds(d0, DW)])


@jax.jit
def kernel(val):
    mesh = plsc.VectorSubcoreMesh(core_axis_name="c", subcore_axis_name="s")
    f = pl.kernel(
        _cumsum_body,
        out_type=jax.ShapeDtypeStruct((B, S, D), jnp.float32),
        mesh=mesh,
        scratch_types=[
            pltpu.VMEM((S_CHUNK, DW), jnp.float32),
            pltpu.VMEM((S_CHUNK, DW), jnp.float32),
        ],
    )
    return f(val)


# speedup vs baseline: 2.8997x; 1.5484x over previous
import jax
import jax.numpy as jnp
from jax import lax
from jax.experimental import pallas as pl
from jax.experimental.pallas import tpu as pltpu
from jax.experimental.pallas import tpu_sc as plsc

B, S, D = 4, 4096, 2048
NC, NS = 2, 16
NW = NC * NS
DW = 256
ND = D // DW
NV = DW // 16
S_CHUNK = 64
N_CHUNK = S // S_CHUNK


def _scan_chunk(in_ref, out_ref, carry):
    def row(s, carry):
        new = []
        for j in range(NV):
            x = in_ref[s, pl.ds(j * 16, 16)]
            acc = carry[j] + x
            out_ref[s, pl.ds(j * 16, 16)] = acc
            new.append(acc)
        return tuple(new)

    return lax.fori_loop(0, S_CHUNK, row, carry)


def _cumsum_body(val_hbm, out_hbm, in0, in1, o0, o1, si0, si1, so0, so1):
    wid = lax.axis_index("s") * NC + lax.axis_index("c")
    b = wid // ND
    d0 = pl.multiple_of((wid % ND) * DW, DW)

    def src(c):
        s0 = pl.multiple_of(c * S_CHUNK, S_CHUNK)
        return val_hbm.at[b, pl.ds(s0, S_CHUNK), pl.ds(d0, DW)]

    def dst(c):
        s0 = pl.multiple_of(c * S_CHUNK, S_CHUNK)
        return out_hbm.at[b, pl.ds(s0, S_CHUNK), pl.ds(d0, DW)]

    pltpu.async_copy(src(0), in0, si0)
    carry0 = tuple(jnp.zeros((16,), jnp.float32) for _ in range(NV))

    def pair(i, carry):
        c0 = 2 * i
        pltpu.async_copy(src(c0 + 1), in1, si1)
        pltpu.make_async_copy(src(c0), in0, si0).wait()

        @pl.when(i > 0)
        def _():
            pltpu.make_async_copy(o0, dst(c0 - 2), so0).wait()

        carry = _scan_chunk(in0, o0, carry)
        pltpu.async_copy(o0, dst(c0), so0)

        @pl.when(c0 + 2 < N_CHUNK)
        def _():
            pltpu.async_copy(src(c0 + 2), in0, si0)

        pltpu.make_async_copy(src(c0 + 1), in1, si1).wait()

        @pl.when(i > 0)
        def _():
            pltpu.make_async_copy(o1, dst(c0 - 1), so1).wait()

        carry = _scan_chunk(in1, o1, carry)
        pltpu.async_copy(o1, dst(c0 + 1), so1)
        return carry

    lax.fori_loop(0, N_CHUNK // 2, pair, carry0)
    pltpu.make_async_copy(o0, dst(N_CHUNK - 2), so0).wait()
    pltpu.make_async_copy(o1, dst(N_CHUNK - 1), so1).wait()


@jax.jit
def kernel(val):
    mesh = plsc.VectorSubcoreMesh(core_axis_name="c", subcore_axis_name="s")
    f = pl.kernel(
        _cumsum_body,
        out_type=jax.ShapeDtypeStruct((B, S, D), jnp.float32),
        mesh=mesh,
        scratch_types=[
            pltpu.VMEM((S_CHUNK, DW), jnp.float32),
            pltpu.VMEM((S_CHUNK, DW), jnp.float32),
            pltpu.VMEM((S_CHUNK, DW), jnp.float32),
            pltpu.VMEM((S_CHUNK, DW), jnp.float32),
            pltpu.SemaphoreType.DMA,
            pltpu.SemaphoreType.DMA,
            pltpu.SemaphoreType.DMA,
            pltpu.SemaphoreType.DMA,
        ],
    )
    return f(val)
